# Initial kernel scaffold; baseline (speedup 1.0000x reference)
import jax
import jax.numpy as jnp
from jax.experimental import pallas as pl


def kernel(x, edge_index, node_region, params):
    raise NotImplementedError("write your pallas kernel here")


# speedup vs baseline: 2.6749x; 2.6749x over previous
import functools

import jax
import jax.numpy as jnp
from jax import lax
from jax.experimental import pallas as pl
from jax.experimental.pallas import tpu as pltpu
from jax.experimental.pallas import tpu_sc as plsc

NC, NS, LANES = 2, 16, 16
NW = NC * NS
EC = 80
RC = 40


def _sc_mesh():
    return plsc.VectorSubcoreMesh(
        core_axis_name="c", subcore_axis_name="s", num_cores=NC, num_subcores=NS)


def _wid():
    return lax.axis_index("s") * NC + lax.axis_index("c")


def _make_prep(N, NP, E, R3):
    ECP = 40
    EW = E // NW
    NCH = EW // ECP
    NPT = NP // NS
    NRCH = N // RC

    @functools.partial(
        pl.kernel, mesh=_sc_mesh(),
        out_type=(
            jax.ShapeDtypeStruct((E, 16), jnp.float32),
            jax.ShapeDtypeStruct((NC, NP, 128), jnp.float32),
            jax.ShapeDtypeStruct((NP, R3), jnp.float32),
        ),
        scratch_types=[
            pltpu.VMEM_SHARED((NP, 128), jnp.float32),
            pltpu.VMEM((ECP,), jnp.int32),
            pltpu.VMEM((ECP,), jnp.int32),
            pltpu.VMEM((ECP, 128), jnp.float32),
            pltpu.VMEM((ECP, 128), jnp.float32),
            pltpu.VMEM((ECP, 16), jnp.float32),
            pltpu.VMEM((ECP, 128), jnp.float32),
            pltpu.VMEM((RC,), jnp.int32),
            pltpu.VMEM((RC, R3), jnp.float32),
            pltpu.SemaphoreType.DMA,
            pltpu.SemaphoreType.DMA,
        ],
    )
    def prep(posp, srci, dsti, regioni, regcat, zerosd,
             rel_o, degp_o, regb_o,
             dacc, sidx, didx, ps, pd, rel, ones, ridx, rrows, sem1, sem2):
        cid = lax.axis_index("c")
        sid = lax.axis_index("s")
        wid = _wid()

        def fill_ones(i, _):
            for cv in range(8):
                ones[i, pl.ds(cv * LANES, LANES)] = jnp.ones((LANES,),
                                                             jnp.float32)
            return 0
        lax.fori_loop(0, ECP, fill_ones, 0)

        pltpu.sync_copy(zerosd.at[pl.ds(sid * NPT, NPT)],
                        dacc.at[pl.ds(sid * NPT, NPT)])
        plsc.subcore_barrier()

        def echunk(j, _):
            base = wid * EW + j * ECP
            pltpu.sync_copy(srci.at[pl.ds(base, ECP)], sidx)
            pltpu.sync_copy(dsti.at[pl.ds(base, ECP)], didx)
            c1 = pltpu.async_copy(posp.at[sidx], ps, sem1)
            c2 = pltpu.async_copy(posp.at[didx], pd, sem2)
            c1.wait()
            c2.wait()

            def row(i, _):
                rel[i, :] = ps[i, pl.ds(0, LANES)] - pd[i, pl.ds(0, LANES)]
                return 0
            lax.fori_loop(0, ECP, row, 0)
            pltpu.sync_copy(rel, rel_o.at[pl.ds(base, ECP)])
            pltpu.sync_copy(ones, dacc.at[didx], add=True)
            return 0
        lax.fori_loop(0, NCH, echunk, 0)

        def rchunk(k, _):
            g = wid + k * NW

            @pl.when(g < NRCH)
            def _():
                b = g * RC
                pltpu.sync_copy(regioni.at[pl.ds(b, RC)], ridx)
                pltpu.async_copy(regcat.at[ridx], rrows, sem1).wait()
                pltpu.sync_copy(rrows, regb_o.at[pl.ds(b, RC)])
            return 0
        lax.fori_loop(0, pl.cdiv(NRCH, NW), rchunk, 0)

        plsc.subcore_barrier()
        pltpu.sync_copy(dacc.at[pl.ds(sid * NPT, NPT)],
                        degp_o.at[cid, pl.ds(sid * NPT, NPT)])

    return prep


def _make_agg(NP, E, D):
    EW = E // NW
    NCH = EW // EC
    NPT = NP // NS

    @functools.partial(
        pl.kernel, mesh=_sc_mesh(),
        out_type=jax.ShapeDtypeStruct((NC, NP, D), jnp.float32),
        scratch_types=[
            pltpu.VMEM_SHARED((NP, D), jnp.float32),
            pltpu.VMEM((EC,), jnp.int32),
            pltpu.VMEM((EC,), jnp.int32),
            pltpu.VMEM((EC, D), jnp.float32),
            pltpu.VMEM((EC, D), jnp.float32),
            pltpu.SemaphoreType.DMA,
            pltpu.SemaphoreType.DMA,
        ],
    )
    def agg(hx, ker, srci, dsti, zerosd, out_o,
            acc, sidx, didx, gbuf, kbuf, sem1, sem2):
        cid = lax.axis_index("c")
        sid = lax.axis_index("s")
        wid = _wid()

        pltpu.sync_copy(zerosd.at[pl.ds(sid * NPT, NPT)],
                        acc.at[pl.ds(sid * NPT, NPT)])
        plsc.subcore_barrier()

        def chunk(j, _):
            base = wid * EW + j * EC
            pltpu.sync_copy(srci.at[pl.ds(base, EC)], sidx)
            pltpu.sync_copy(dsti.at[pl.ds(base, EC)], didx)
            c1 = pltpu.async_copy(hx.at[sidx], gbuf, sem1)
            c2 = pltpu.async_copy(ker.at[pl.ds(base, EC)], kbuf, sem2)
            c1.wait()
            c2.wait()

            def row(i, _):
                for cv in range(D // LANES):
                    sl = pl.ds(cv * LANES, LANES)
                    gbuf[i, sl] = gbuf[i, sl] * kbuf[i, sl]
                return 0
            lax.fori_loop(0, EC, row, 0)
            pltpu.sync_copy(gbuf, acc.at[didx], add=True)
            return 0
        lax.fori_loop(0, NCH, chunk, 0)

        plsc.subcore_barrier()
        pltpu.sync_copy(acc.at[pl.ds(sid * NPT, NPT)],
                        out_o.at[cid, pl.ds(sid * NPT, NPT)])

    return agg


def _make_decgather(NP, E, D):
    EW = E // NW
    NCH = EW // EC

    @functools.partial(
        pl.kernel, mesh=_sc_mesh(),
        out_type=jax.ShapeDtypeStruct((E, D), jnp.float32),
        scratch_types=[
            pltpu.VMEM((EC,), jnp.int32),
            pltpu.VMEM((EC,), jnp.int32),
            pltpu.VMEM((EC, D), jnp.float32),
            pltpu.VMEM((EC, D), jnp.float32),
            pltpu.SemaphoreType.DMA,
            pltpu.SemaphoreType.DMA,
        ],
    )
    def decg(ha, hb, srci, dsti, out_o, sidx, didx, abuf, bbuf, sem1, sem2):
        wid = _wid()

        def chunk(j, _):
            base = wid * EW + j * EC
            pltpu.sync_copy(srci.at[pl.ds(base, EC)], sidx)
            pltpu.sync_copy(dsti.at[pl.ds(base, EC)], didx)
            c1 = pltpu.async_copy(ha.at[sidx], abuf, sem1)
            c2 = pltpu.async_copy(hb.at[didx], bbuf, sem2)
            c1.wait()
            c2.wait()

            def row(i, _):
                for cv in range(D // LANES):
                    sl = pl.ds(cv * LANES, LANES)
                    abuf[i, sl] = abuf[i, sl] + bbuf[i, sl]
                return 0
            lax.fori_loop(0, EC, row, 0)
            pltpu.sync_copy(abuf, out_o.at[pl.ds(base, EC)])
            return 0
        lax.fori_loop(0, NCH, chunk, 0)

    return decg


def _edge_mlp(rel, w1s, b1s, w2s, b2s):
    E = rel.shape[0]
    D = w2s[0].shape[1]
    BE = 3200
    G = E // BE

    def body(rel_ref, *refs):
        xm = rel_ref[...]
        for l in range(3):
            w1, b1, w2, b2 = refs[4 * l:4 * l + 4]
            o = refs[12 + l]
            k1 = jnp.maximum(
                jnp.dot(xm, w1[...], preferred_element_type=jnp.float32)
                + b1[0:1, :], 0.0)
            o[...] = (jnp.dot(k1, w2[...], preferred_element_type=jnp.float32)
                      + b2[0:1, :])

    wspecs = []
    wargs = []
    for l in range(3):
        wargs += [w1s[l], b1s[l], w2s[l], b2s[l]]
        wspecs += [
            pl.BlockSpec(w1s[l].shape, lambda i: (0, 0)),
            pl.BlockSpec(b1s[l].shape, lambda i: (0, 0)),
            pl.BlockSpec(w2s[l].shape, lambda i: (0, 0)),
            pl.BlockSpec(b2s[l].shape, lambda i: (0, 0)),
        ]
    return pl.pallas_call(
        body,
        grid=(G,),
        in_specs=[pl.BlockSpec((BE, 16), lambda i: (i, 0))] + wspecs,
        out_specs=[pl.BlockSpec((BE, D), lambda i: (i, 0))] * 3,
        out_shape=[jax.ShapeDtypeStruct((E, D), jnp.float32)] * 3,
    )(rel, *wargs)


def _matmul(h, w):
    NPR, D = h.shape
    D2 = w.shape[1]
    BN = NPR // 8
    G = NPR // BN

    def body(h_ref, w_ref, o_ref):
        o_ref[...] = jnp.dot(h_ref[...], w_ref[...],
                             preferred_element_type=jnp.float32)

    return pl.pallas_call(
        body,
        grid=(G,),
        in_specs=[pl.BlockSpec((BN, D), lambda i: (i, 0)),
                  pl.BlockSpec((D, D2), lambda i: (0, 0))],
        out_specs=pl.BlockSpec((BN, D2), lambda i: (i, 0)),
        out_shape=jax.ShapeDtypeStruct((NPR, D2), jnp.float32),
    )(h, w)


def _node_finalize(p0, p1, d0, d1, rb, bvec, ws, badd):
    NPR, D = p0.shape
    BN = NPR // 8
    G = NPR // BN
    nouts = len(ws)

    def body(p0r, p1r, d0r, d1r, rbr, br, *rest):
        wrefs = rest[:nouts]
        baddr = rest[nouts:2 * nouts]
        orefs = rest[2 * nouts:]
        deg = d0r[:, 0:1] + d1r[:, 0:1]
        rdeg = 1.0 / jnp.maximum(deg, 1.0)
        h = jnp.maximum((p0r[...] + p1r[...]) * rdeg + rbr[...] + br[0:1, :],
                        0.0)
        for i in range(nouts):
            o = jnp.dot(h, wrefs[i][...], preferred_element_type=jnp.float32)
            orefs[i][...] = o + baddr[i][0:1, :]

    in_specs = [
        pl.BlockSpec((BN, D), lambda i: (i, 0)),
        pl.BlockSpec((BN, D), lambda i: (i, 0)),
        pl.BlockSpec((BN, D), lambda i: (i, 0)),
        pl.BlockSpec((BN, D), lambda i: (i, 0)),
        pl.BlockSpec((BN, D), lambda i: (i, 0)),
        pl.BlockSpec(bvec.shape, lambda i: (0, 0)),
    ]
    args = [p0, p1, d0, d1, rb, bvec]
    for w in ws:
        in_specs.append(pl.BlockSpec(w.shape, lambda i: (0, 0)))
        args.append(w)
    for ba in badd:
        in_specs.append(pl.BlockSpec(ba.shape, lambda i: (0, 0)))
        args.append(ba)
    outs = pl.pallas_call(
        body,
        grid=(G,),
        in_specs=in_specs,
        out_specs=[pl.BlockSpec((BN, w.shape[1]), lambda i: (i, 0))
                   for w in ws],
        out_shape=[jax.ShapeDtypeStruct((NPR, w.shape[1]), jnp.float32)
                   for w in ws],
    )(*args)
    return outs


def _dec_mlp(z1pre, w2, b2, w3row, b3):
    E, D = z1pre.shape
    BE = 3200
    G = E // BE

    def body(z_ref, w2r, b2r, w3r, b3r, o_ref):
        z1 = jnp.maximum(z_ref[...], 0.0)
        z2 = jnp.maximum(
            jnp.dot(z1, w2r[...], preferred_element_type=jnp.float32)
            + b2r[0:1, :], 0.0)
        s = jnp.sum(z2 * w3r[0:1, :], axis=1, keepdims=True) + b3r[0:1, 0:1]
        o_ref[...] = jax.nn.sigmoid(s)

    return pl.pallas_call(
        body,
        grid=(G,),
        in_specs=[pl.BlockSpec((BE, D), lambda i: (i, 0)),
                  pl.BlockSpec(w2.shape, lambda i: (0, 0)),
                  pl.BlockSpec(b2.shape, lambda i: (0, 0)),
                  pl.BlockSpec(w3row.shape, lambda i: (0, 0)),
                  pl.BlockSpec(b3.shape, lambda i: (0, 0))],
        out_specs=pl.BlockSpec((BE, 1), lambda i: (i, 0)),
        out_shape=jax.ShapeDtypeStruct((E, 1), jnp.float32),
    )(z1pre, w2, b2, w3row, b3)


def _b8(b):
    return jnp.broadcast_to(b[None, :], (8, b.shape[0]))


def kernel(x, edge_index, node_region, params):
    N = x.shape[0]
    D = x.shape[1] - 2
    E = edge_index.shape[1]

    NP = ((N + 127) // 128) * 128
    pos = x[:, :2]
    h0 = jnp.pad(x[:, 2:], ((0, NP - N), (0, 0)))
    posp = jnp.pad(pos, ((0, 0), (0, 126)))
    src = edge_index[0]
    dst = edge_index[1]
    regcat = jnp.concatenate(
        [params[f"conv{l}_reg"] for l in range(3)], axis=1)
    zerosd = jnp.zeros((NP, D), jnp.float32)

    rel, degp, regb = _make_prep(N, NP, E, 3 * D)(
        posp, src, dst, node_region, regcat, zerosd)

    w1s = [jnp.pad(params[f"conv{l}_Wp1"], ((0, 14), (0, 0))) for l in range(3)]
    b1s = [_b8(params[f"conv{l}_bp1"]) for l in range(3)]
    w2s = [params[f"conv{l}_Wp2"] for l in range(3)]
    b2s = [_b8(params[f"conv{l}_bp2"]) for l in range(3)]
    kers = _edge_mlp(rel, w1s, b1s, w2s, b2s)

    hx = _matmul(h0, params["conv0_Wx"])
    d0 = degp[0]
    d1 = degp[1]
    agg_call = _make_agg(NP, E, D)
    zb = _b8(jnp.zeros((D,), jnp.float32))

    ha = hb = None
    for l in range(3):
        p = agg_call(hx, kers[l], src, dst, zerosd)
        rb = regb[:, l * D:(l + 1) * D]
        bvec = _b8(params[f"conv{l}_b"])
        if l < 2:
            (hx,) = _node_finalize(p[0], p[1], d0, d1, rb, bvec,
                                   [params[f"conv{l + 1}_Wx"]], [zb])
        else:
            w1a = params["dec_W1"][:D]
            w1b = params["dec_W1"][D:]
            ha, hb = _node_finalize(p[0], p[1], d0, d1, rb, bvec,
                                    [w1a, w1b], [_b8(params["dec_b1"]), zb])

    z1pre = _make_decgather(NP, E, D)(ha, hb, src, dst)
    w3row = _b8(params["dec_W3"][:, 0])
    b3 = jnp.broadcast_to(params["dec_b3"][None, :], (8, 8))
    return _dec_mlp(z1pre, params["dec_W2"], _b8(params["dec_b2"]), w3row, b3)
